# baseline (speedup 1.0000x reference)
import jax
import jax.numpy as jnp
from jax.experimental import pallas as pl
from jax.experimental.pallas import tpu as pltpu

_K = 1024
_D = 64
_COMMIT = 0.25
_BP = 1024


def _vq_kernel(z2_ref, w_ref, q2_ref, loss_ref, ppl_ref, counts_ref, sq_ref):
    i = pl.program_id(0)
    nblk = pl.num_programs(0)
    z2 = z2_ref[...]
    w = w_ref[...]

    zpad = jnp.zeros((_K, _D), jnp.float32)
    rw = jnp.concatenate([
        jnp.concatenate([w, zpad], axis=1),
        jnp.concatenate([zpad, w], axis=1),
    ], axis=0)
    rm2 = rw * (-2.0)

    s2 = jax.lax.dot_general(
        z2, rm2, (((1,), (1,)), ((), ())), preferred_element_type=jnp.float32)
    ze = z2[:, :_D]
    zo = z2[:, _D:]
    zsqe = jnp.sum(ze * ze, axis=1, keepdims=True)
    zsqo = jnp.sum(zo * zo, axis=1, keepdims=True)
    wsq = jnp.sum(w * w, axis=1)
    base = jnp.concatenate(
        [zsqe + wsq[None, :], zsqo + wsq[None, :]], axis=1)
    d = base + s2

    idxe = jnp.argmin(d[:, :_K], axis=1).astype(jnp.int32)
    idxo = jnp.argmin(d[:, _K:], axis=1).astype(jnp.int32)

    col = jax.lax.broadcasted_iota(jnp.int32, (_BP, _K), 1)
    onehot = jnp.concatenate(
        [(col == idxe[:, None]), (col == idxo[:, None])],
        axis=1).astype(jnp.float32)
    q2 = jax.lax.dot_general(
        onehot, rw, (((1,), (0,)), ((), ())),
        preferred_element_type=jnp.float32)
    q2_ref[...] = q2

    ones_row = jnp.ones((1, _BP), jnp.float32)
    cnt2 = jax.lax.dot_general(
        ones_row, onehot, (((1,), (0,)), ((), ())),
        preferred_element_type=jnp.float32)
    blk_counts = cnt2[:, :_K] + cnt2[:, _K:]
    diff = q2 - z2
    blk_sq = jnp.sum(diff * diff)

    @pl.when(i == 0)
    def _init():
        counts_ref[...] = blk_counts
        sq_ref[0, 0] = blk_sq

    @pl.when(i > 0)
    def _acc():
        counts_ref[...] += blk_counts
        sq_ref[0, 0] += blk_sq

    @pl.when(i == nblk - 1)
    def _final():
        n_total = nblk * _BP * 2
        mse = sq_ref[0, 0] / jnp.float32(n_total * _D)
        loss_ref[...] = jnp.full((1, 1), (1.0 + _COMMIT) * mse, jnp.float32)
        p = counts_ref[...] / jnp.float32(n_total)
        ent = -jnp.sum(p * jnp.log(p + 1e-10))
        ppl_ref[...] = jnp.full((1, 1), jnp.exp(ent), jnp.float32)


def kernel(inputs, W):
    n = inputs.shape[0]
    z2 = inputs.reshape(n // 2, 2 * _D)
    grid = (n // 2 // _BP,)
    q2, loss, ppl = pl.pallas_call(
        _vq_kernel,
        grid=grid,
        in_specs=[
            pl.BlockSpec((_BP, 2 * _D), lambda i: (i, 0)),
            pl.BlockSpec((_K, _D), lambda i: (0, 0)),
        ],
        out_specs=[
            pl.BlockSpec((_BP, 2 * _D), lambda i: (i, 0)),
            pl.BlockSpec((1, 1), lambda i: (0, 0)),
            pl.BlockSpec((1, 1), lambda i: (0, 0)),
        ],
        out_shape=[
            jax.ShapeDtypeStruct((n // 2, 2 * _D), jnp.float32),
            jax.ShapeDtypeStruct((1, 1), jnp.float32),
            jax.ShapeDtypeStruct((1, 1), jnp.float32),
        ],
        scratch_shapes=[
            pltpu.VMEM((1, _K), jnp.float32),
            pltpu.SMEM((1, 1), jnp.float32),
        ],
    )(z2, W)
    q = q2.reshape(n, _D)
    return q, loss[0, 0], ppl[0, 0]


# speedup vs baseline: 1.2177x; 1.2177x over previous
import jax
import jax.numpy as jnp
from jax.experimental import pallas as pl
from jax.experimental.pallas import tpu as pltpu

_K = 1024
_D = 64
_COMMIT = 0.25
_BN = 2048


def _vq_kernel(z_ref, w_ref, q_ref, loss_ref, ppl_ref, counts_ref, sq_ref):
    i = pl.program_id(0)
    nblk = pl.num_programs(0)
    z = z_ref[...]
    w = w_ref[...]

    wm2 = w * (-2.0)
    s2 = jax.lax.dot_general(
        z, wm2, (((1,), (1,)), ((), ())), preferred_element_type=jnp.float32)
    zsq = jnp.sum(z * z, axis=1, keepdims=True)
    wsq = jnp.sum(w * w, axis=1)
    d = (zsq + wsq[None, :]) + s2

    idx = jnp.argmin(d, axis=1).astype(jnp.int32)

    col = jax.lax.broadcasted_iota(jnp.int32, d.shape, 1)
    onehot = (col == idx[:, None]).astype(jnp.float32)
    q = jax.lax.dot_general(
        onehot, w, (((1,), (0,)), ((), ())), preferred_element_type=jnp.float32)
    q_ref[...] = q

    ones_row = jnp.ones((1, _BN), jnp.float32)
    blk_counts = jax.lax.dot_general(
        ones_row, onehot, (((1,), (0,)), ((), ())),
        preferred_element_type=jnp.float32)
    diff = q - z
    blk_sq = jnp.sum(diff * diff)

    @pl.when(i == 0)
    def _init():
        counts_ref[...] = blk_counts
        sq_ref[0, 0] = blk_sq

    @pl.when(i > 0)
    def _acc():
        counts_ref[...] += blk_counts
        sq_ref[0, 0] += blk_sq

    @pl.when(i == nblk - 1)
    def _final():
        n_total = (nblk * _BN)
        mse = sq_ref[0, 0] / jnp.float32(n_total * _D)
        loss_ref[...] = jnp.full((1, 1), (1.0 + _COMMIT) * mse, jnp.float32)
        p = counts_ref[...] / jnp.float32(n_total)
        ent = -jnp.sum(p * jnp.log(p + 1e-10))
        ppl_ref[...] = jnp.full((1, 1), jnp.exp(ent), jnp.float32)


def kernel(inputs, W):
    n = inputs.shape[0]
    grid = (n // _BN,)
    q, loss, ppl = pl.pallas_call(
        _vq_kernel,
        grid=grid,
        in_specs=[
            pl.BlockSpec((_BN, _D), lambda i: (i, 0)),
            pl.BlockSpec((_K, _D), lambda i: (0, 0)),
        ],
        out_specs=[
            pl.BlockSpec((_BN, _D), lambda i: (i, 0)),
            pl.BlockSpec((1, 1), lambda i: (0, 0)),
            pl.BlockSpec((1, 1), lambda i: (0, 0)),
        ],
        out_shape=[
            jax.ShapeDtypeStruct((n, _D), jnp.float32),
            jax.ShapeDtypeStruct((1, 1), jnp.float32),
            jax.ShapeDtypeStruct((1, 1), jnp.float32),
        ],
        scratch_shapes=[
            pltpu.VMEM((1, _K), jnp.float32),
            pltpu.SMEM((1, 1), jnp.float32),
        ],
    )(inputs, W)
    return q, loss[0, 0], ppl[0, 0]
